# Initial kernel scaffold; baseline (speedup 1.0000x reference)
import jax
import jax.numpy as jnp
from jax.experimental import pallas as pl


def kernel(pred_kps, target_kps, stride_tensor, target_scores, target_scores_sum, fg_mask):
    raise NotImplementedError("write your pallas kernel here")


# speedup vs baseline: 3.3714x; 3.3714x over previous
import functools

import jax
import jax.numpy as jnp
from jax import lax
from jax.experimental import pallas as pl
from jax.experimental.pallas import tpu as pltpu
from jax.experimental.pallas import tpu_sc as plsc

NC = 2
NS = 16
L = 16
NW = NC * NS

_TOFF = (0, 1, 3, 4, 6, 7, 9, 10, 12, 13)


def _sc_loss_body(R, NA, T, NT, pred, targ, score, mask, stride, out,
                  p0, p1, t0, t1, s0, s1, m0, m1, st0, st1, ostage, *sems):
    RW = R // NW
    B = T // L
    wid = lax.axis_index("s") * NC + lax.axis_index("c")
    row0 = wid * RW
    srow0 = lax.rem(row0, NA)

    bufs = ((p0, t0, s0, m0, st0), (p1, t1, s1, m1, st1))

    def start(i, b):
        pv, tv, sv, mv, stv = bufs[b]
        sm = sems[5 * b:5 * b + 5]
        r = i * T
        return (
            pltpu.async_copy(pred.at[pl.ds((row0 + r) * 10, T * 10)], pv, sm[0]),
            pltpu.async_copy(targ.at[pl.ds((row0 + r) * 15, T * 15)], tv, sm[1]),
            pltpu.async_copy(score.at[pl.ds(row0 + r, T)], sv, sm[2]),
            pltpu.async_copy(mask.at[pl.ds(row0 + r, T)], mv, sm[3]),
            pltpu.async_copy(stride.at[pl.ds(srow0 + r, T)], stv, sm[4]),
        )

    io10 = lax.iota(jnp.int32, L) * 10
    io15 = lax.iota(jnp.int32, L) * 15

    def tile_compute(b, accl, accn):
        pv, tv, sv, mv, stv = bufs[b]

        def block(j, carry):
            al, an = carry
            r0 = j * L
            sc = sv[pl.ds(r0, L)]
            mk = mv[pl.ds(r0, L)]
            st = stv[pl.ds(r0, L)]
            w = sc * mk
            si = 1.0 / st
            an = an + mk
            pb = r0 * 10
            tb = r0 * 15
            for f in range(10):
                p = plsc.load_gather(pv, [io10 + (pb + f)])
                t = plsc.load_gather(tv, [io15 + (tb + _TOFF[f])])
                d = jnp.abs(p - t * si)
                m = jnp.minimum(d, 1.0)
                al = al + (d - 0.5 * m) * m * w
            return al, an

        return lax.fori_loop(0, B, block, (accl, accn))

    accl = jnp.zeros((L,), jnp.float32)
    accn = jnp.zeros((L,), jnp.float32)
    handles = {0: start(0, 0)}
    for i in range(NT):
        if i + 1 < NT:
            handles[i + 1] = start(i + 1, (i + 1) % 2)
        for h in handles.pop(i):
            h.wait()
        accl, accn = tile_compute(i % 2, accl, accn)

    ostage[pl.ds(0, L)] = accl
    ostage[pl.ds(L, L)] = accn
    pltpu.sync_copy(ostage, out.at[pl.ds(wid * 2 * L, 2 * L)])


@functools.partial(jax.jit, static_argnums=(5, 6, 7, 8))
def _sc_loss(pred, targ, score, mask, stride, R, NA, T, NT):
    mesh = plsc.VectorSubcoreMesh(core_axis_name="c", subcore_axis_name="s",
                                  num_cores=NC, num_subcores=NS)
    body = functools.partial(_sc_loss_body, R, NA, T, NT)
    f = pl.kernel(
        body,
        out_type=jax.ShapeDtypeStruct((NW * 2 * L,), jnp.float32),
        mesh=mesh,
        scratch_types=[
            pltpu.VMEM((T * 10,), jnp.float32),
            pltpu.VMEM((T * 10,), jnp.float32),
            pltpu.VMEM((T * 15,), jnp.float32),
            pltpu.VMEM((T * 15,), jnp.float32),
            pltpu.VMEM((T,), jnp.float32),
            pltpu.VMEM((T,), jnp.float32),
            pltpu.VMEM((T,), jnp.float32),
            pltpu.VMEM((T,), jnp.float32),
            pltpu.VMEM((T,), jnp.float32),
            pltpu.VMEM((T,), jnp.float32),
            pltpu.VMEM((2 * L,), jnp.float32),
        ] + [pltpu.SemaphoreType.DMA] * 10,
        compiler_params=pltpu.CompilerParams(needs_layout_passes=False),
    )
    return f(pred, targ, score, mask, stride)


def kernel(pred_kps, target_kps, stride_tensor, target_scores,
           target_scores_sum, fg_mask):
    bs, na = fg_mask.shape
    R = bs * na
    RW = R // NW
    T = 1680
    NT = RW // T

    pred = pred_kps.reshape(-1)
    targ = target_kps.reshape(-1)
    score = target_scores.reshape(-1)
    mask = fg_mask.astype(jnp.float32).reshape(-1)
    stride = stride_tensor.reshape(-1)

    o = _sc_loss(pred, targ, score, mask, stride, R, na, T, NT).reshape(NW, 2, L)
    loss_sum = o[:, 0].sum()
    num_pos = o[:, 1].sum()
    denom = num_pos * 10.0
    safe = jnp.where(denom == 0.0, jnp.float32(1.0), denom)
    l = loss_sum / safe
    ts = target_scores_sum.reshape(())
    lpos = jnp.where(ts == 0.0, l, l / ts)
    return jnp.where(num_pos > 0.0, lpos, jnp.float32(0.0))


# speedup vs baseline: 18.9500x; 5.6208x over previous
import functools

import jax
import jax.numpy as jnp
from jax import lax
from jax.experimental import pallas as pl
from jax.experimental.pallas import tpu as pltpu
from jax.experimental.pallas import tpu_sc as plsc

NC = 2
NS = 16
L = 16
NW = NC * NS

_TOFF = (0, 1, 3, 4, 6, 7, 9, 10, 12, 13)
NF = 10


def _sc_loss_body(R, NA, T, NT, pred, targ, score, mask, stride, out,
                  pb0, pb1, tb0, tb1, s0, s1, m0, m1, st0, st1, ostage,
                  sem0, sem1):
    RW = R // NW
    B = T // L
    wid = lax.axis_index("s") * NC + lax.axis_index("c")
    a0 = wid * RW
    sa0 = lax.rem(a0, NA)

    bufs = ((pb0, tb0, s0, m0, st0, sem0), (pb1, tb1, s1, m1, st1, sem1))

    def start(i, b):
        pv, tv, sv, mv, stv, sm = bufs[b]
        r = a0 + i * T
        hs = []
        for f in range(NF):
            hs.append(pltpu.async_copy(
                pred.at[pl.ds(f * R + r, T)], pv.at[pl.ds(f * T, T)], sm))
            hs.append(pltpu.async_copy(
                targ.at[pl.ds(_TOFF[f] * R + r, T)], tv.at[pl.ds(f * T, T)], sm))
        hs.append(pltpu.async_copy(score.at[pl.ds(r, T)], sv, sm))
        hs.append(pltpu.async_copy(mask.at[pl.ds(r, T)], mv, sm))
        hs.append(pltpu.async_copy(stride.at[pl.ds(sa0 + i * T, T)], stv, sm))
        return hs

    def tile_compute(b, accl, accn):
        pv, tv, sv, mv, stv, _ = bufs[b]

        def chunk(j, carry):
            al, an = carry
            o = j * L
            mk = mv[pl.ds(o, L)]
            w = sv[pl.ds(o, L)] * mk
            si = 1.0 / stv[pl.ds(o, L)]
            an = an + mk
            for f in range(NF):
                p = pv[pl.ds(f * T + o, L)]
                t = tv[pl.ds(f * T + o, L)]
                d = jnp.abs(p - t * si)
                m = jnp.minimum(d, 1.0)
                al = al + (d - 0.5 * m) * m * w
            return al, an

        return lax.fori_loop(0, B, chunk, (accl, accn))

    accl = jnp.zeros((L,), jnp.float32)
    accn = jnp.zeros((L,), jnp.float32)
    handles = {0: start(0, 0)}
    for i in range(NT):
        if i + 1 < NT:
            handles[i + 1] = start(i + 1, (i + 1) % 2)
        for h in handles.pop(i):
            h.wait()
        accl, accn = tile_compute(i % 2, accl, accn)

    ostage[pl.ds(0, L)] = accl
    ostage[pl.ds(L, L)] = accn
    pltpu.sync_copy(ostage, out.at[pl.ds(wid * 2 * L, 2 * L)])


@functools.partial(jax.jit, static_argnums=(5, 6, 7, 8))
def _sc_loss(pred, targ, score, mask, stride, R, NA, T, NT):
    mesh = plsc.VectorSubcoreMesh(core_axis_name="c", subcore_axis_name="s",
                                  num_cores=NC, num_subcores=NS)
    body = functools.partial(_sc_loss_body, R, NA, T, NT)
    f = pl.kernel(
        body,
        out_type=jax.ShapeDtypeStruct((NW * 2 * L,), jnp.float32),
        mesh=mesh,
        scratch_types=[
            pltpu.VMEM((NF * T,), jnp.float32),
            pltpu.VMEM((NF * T,), jnp.float32),
            pltpu.VMEM((NF * T,), jnp.float32),
            pltpu.VMEM((NF * T,), jnp.float32),
            pltpu.VMEM((T,), jnp.float32),
            pltpu.VMEM((T,), jnp.float32),
            pltpu.VMEM((T,), jnp.float32),
            pltpu.VMEM((T,), jnp.float32),
            pltpu.VMEM((T,), jnp.float32),
            pltpu.VMEM((T,), jnp.float32),
            pltpu.VMEM((2 * L,), jnp.float32),
            pltpu.SemaphoreType.DMA,
            pltpu.SemaphoreType.DMA,
        ],
        compiler_params=pltpu.CompilerParams(needs_layout_passes=False),
    )
    return f(pred, targ, score, mask, stride)


def kernel(pred_kps, target_kps, stride_tensor, target_scores,
           target_scores_sum, fg_mask):
    bs, na = fg_mask.shape
    R = bs * na
    RW = R // NW
    T = 1680
    NT = RW // T

    pred = pred_kps.transpose(2, 0, 1).reshape(-1)
    targ = target_kps.transpose(2, 0, 1).reshape(-1)
    score = target_scores.reshape(bs, na).reshape(-1)
    mask = fg_mask.astype(jnp.float32).reshape(-1)
    stride = stride_tensor.reshape(-1)

    o = _sc_loss(pred, targ, score, mask, stride, R, na, T, NT).reshape(NW, 2, L)
    loss_sum = o[:, 0].sum()
    num_pos = o[:, 1].sum()
    denom = num_pos * 10.0
    safe = jnp.where(denom == 0.0, jnp.float32(1.0), denom)
    l = loss_sum / safe
    ts = target_scores_sum.reshape(())
    lpos = jnp.where(ts == 0.0, l, l / ts)
    return jnp.where(num_pos > 0.0, lpos, jnp.float32(0.0))


# speedup vs baseline: 19.5592x; 1.0321x over previous
import functools

import jax
import jax.numpy as jnp
from jax import lax
from jax.experimental import pallas as pl
from jax.experimental.pallas import tpu as pltpu
from jax.experimental.pallas import tpu_sc as plsc

NC = 2
NS = 16
L = 16
NW = NC * NS

_TOFF = (0, 1, 3, 4, 6, 7, 9, 10, 12, 13)
NF = 10
TR = 8
TLC = 128


def _sc_loss_body(NA, NB, pred, targ, score, mask, stride, out,
                  pbuf, tbuf, sbuf, mbuf, stbuf, ostage, sem0, sem1):
    nrt = NB // TR
    nct = (NA + TLC - 1) // TLC
    NU = nrt * nct
    SLOTS = (NU // NW) + 2
    wid = lax.axis_index("s") * NC + lax.axis_index("c")
    u0 = (wid * NU) // NW
    cnt = ((wid + 1) * NU) // NW - u0
    sems = (sem0, sem1)

    def unit_of(k):
        u = u0 + jnp.minimum(k, cnt - 1)
        tc = u // nrt
        tr = u - tc * nrt
        return tr * TR, tc * TLC, tc

    def issue(k, b):
        ro, co, _ = unit_of(k)
        sm = sems[b]
        for p in range(NF):
            pltpu.async_copy(pred.at[p, pl.ds(ro, TR), pl.ds(co, TLC)],
                             pbuf.at[b, p], sm)
            pltpu.async_copy(targ.at[_TOFF[p], pl.ds(ro, TR), pl.ds(co, TLC)],
                             tbuf.at[b, p], sm)
        pltpu.async_copy(score.at[pl.ds(ro, TR), pl.ds(co, TLC)], sbuf.at[b], sm)
        pltpu.async_copy(mask.at[pl.ds(ro, TR), pl.ds(co, TLC)], mbuf.at[b], sm)
        pltpu.async_copy(stride.at[pl.ds(co, TLC)], stbuf.at[b], sm)

    def drain(b):
        sm = sems[b]
        for p in range(NF):
            pltpu.make_async_copy(pred.at[0, pl.ds(0, TR), pl.ds(0, TLC)],
                                  pbuf.at[b, p], sm).wait()
            pltpu.make_async_copy(targ.at[0, pl.ds(0, TR), pl.ds(0, TLC)],
                                  tbuf.at[b, p], sm).wait()
        pltpu.make_async_copy(score.at[pl.ds(0, TR), pl.ds(0, TLC)],
                              sbuf.at[b], sm).wait()
        pltpu.make_async_copy(mask.at[pl.ds(0, TR), pl.ds(0, TLC)],
                              mbuf.at[b], sm).wait()
        pltpu.make_async_copy(stride.at[pl.ds(0, TLC)], stbuf.at[b], sm).wait()

    def compute(k, b, al, an):
        _, _, tc = unit_of(k)
        vc = jnp.where(k < cnt,
                       jnp.minimum(NA - tc * TLC, TLC) // L,
                       0)

        def jbody(j, carry):
            al, an = carry
            co = j * L
            si = 1.0 / stbuf[b, pl.ds(co, L)]
            for r in range(TR):
                mk = mbuf[b, r, pl.ds(co, L)]
                w = sbuf[b, r, pl.ds(co, L)] * mk
                an = an + mk
                for p in range(NF):
                    pp = pbuf[b, p, r, pl.ds(co, L)]
                    tt = tbuf[b, p, r, pl.ds(co, L)]
                    d = jnp.abs(pp - tt * si)
                    m = jnp.minimum(d, 1.0)
                    al = al + (d - 0.5 * m) * m * w
            return al, an

        return lax.fori_loop(0, vc, jbody, (al, an))

    issue(jnp.int32(0), 0)

    def pair(kp, carry):
        al, an = carry
        for b in (0, 1):
            k = kp * 2 + b

            @pl.when(k + 1 < SLOTS)
            def _():
                issue(k + 1, 1 - b)

            drain(b)
            al, an = compute(k, b, al, an)
        return al, an

    al = jnp.zeros((L,), jnp.float32)
    an = jnp.zeros((L,), jnp.float32)
    al, an = lax.fori_loop(0, SLOTS // 2, pair, (al, an))

    ostage[pl.ds(0, L)] = al
    ostage[pl.ds(L, L)] = an
    pltpu.sync_copy(ostage, out.at[pl.ds(wid * 2 * L, 2 * L)])


@functools.partial(jax.jit, static_argnums=(5, 6))
def _sc_loss(pred, targ, score, mask, stride, NA, NB):
    mesh = plsc.VectorSubcoreMesh(core_axis_name="c", subcore_axis_name="s",
                                  num_cores=NC, num_subcores=NS)
    body = functools.partial(_sc_loss_body, NA, NB)
    f = pl.kernel(
        body,
        out_type=jax.ShapeDtypeStruct((NW * 2 * L,), jnp.float32),
        mesh=mesh,
        scratch_types=[
            pltpu.VMEM((2, NF, TR, TLC), jnp.float32),
            pltpu.VMEM((2, NF, TR, TLC), jnp.float32),
            pltpu.VMEM((2, TR, TLC), jnp.float32),
            pltpu.VMEM((2, TR, TLC), jnp.float32),
            pltpu.VMEM((2, TLC), jnp.float32),
            pltpu.VMEM((2 * L,), jnp.float32),
            pltpu.SemaphoreType.DMA,
            pltpu.SemaphoreType.DMA,
        ],
        compiler_params=pltpu.CompilerParams(
            needs_layout_passes=False,
            use_tc_tiling_on_sc=True,
            disable_bounds_checks=True,
        ),
    )
    return f(pred, targ, score, mask, stride)


def kernel(pred_kps, target_kps, stride_tensor, target_scores,
           target_scores_sum, fg_mask):
    bs, na = fg_mask.shape

    pred = pred_kps.transpose(2, 0, 1)
    targ = target_kps.transpose(2, 0, 1)
    score = target_scores.reshape(bs, na)
    mask = fg_mask.astype(jnp.float32)
    stride = stride_tensor.reshape(-1)

    o = _sc_loss(pred, targ, score, mask, stride, na, bs).reshape(NW, 2, L)
    loss_sum = o[:, 0].sum()
    num_pos = o[:, 1].sum()
    denom = num_pos * 10.0
    safe = jnp.where(denom == 0.0, jnp.float32(1.0), denom)
    l = loss_sum / safe
    ts = target_scores_sum.reshape(())
    lpos = jnp.where(ts == 0.0, l, l / ts)
    return jnp.where(num_pos > 0.0, lpos, jnp.float32(0.0))
